# baseline (speedup 1.0000x reference)
import functools

import jax
import jax.numpy as jnp
from jax import lax
from jax.experimental import pallas as pl
from jax.experimental.pallas import tpu as pltpu
from jax.experimental.pallas import tpu_sc as plsc

N_EMBD = 64
CHUNK = 100
NW = 32
ROWS_PER_W = 32
T = 200


def _emb_call(tok2d, table, pos2):
    mesh = plsc.VectorSubcoreMesh(core_axis_name="c", subcore_axis_name="s")

    @functools.partial(
        pl.kernel,
        mesh=mesh,
        compiler_params=pltpu.CompilerParams(use_tc_tiling_on_sc=False),
        out_type=jax.ShapeDtypeStruct((NW * ROWS_PER_W, 2, CHUNK, N_EMBD),
                                      jnp.float32),
        scratch_types=[
            pltpu.VMEM((2 * ROWS_PER_W, CHUNK), jnp.int32),
            pltpu.VMEM((2, CHUNK, N_EMBD), jnp.float32),
            pltpu.VMEM((2, CHUNK, N_EMBD), jnp.float32),
            pltpu.SemaphoreType.DMA,
        ],
    )
    def k(tok_hbm, table_hbm, pos_hbm, out_hbm, idx_v, pos_v, buf_v, sem):
        wid = lax.axis_index("s") * 2 + lax.axis_index("c")
        pltpu.sync_copy(tok_hbm.at[pl.ds(wid * 2 * ROWS_PER_W, 2 * ROWS_PER_W)],
                        idx_v)
        pltpu.sync_copy(pos_hbm, pos_v)

        def batch_row(c, carry):
            g0 = pltpu.async_copy(table_hbm.at[idx_v.at[2 * c]],
                                  buf_v.at[0], sem)
            g1 = pltpu.async_copy(table_hbm.at[idx_v.at[2 * c + 1]],
                                  buf_v.at[1], sem)
            g0.wait()
            g1.wait()

            def row_body(r, carry2):
                for h in range(2):
                    for j in range(N_EMBD // 16):
                        sl = pl.ds(j * 16, 16)
                        buf_v[h, r, sl] += pos_v[h, r, sl]
                return carry2

            lax.fori_loop(0, CHUNK, row_body, 0)
            pltpu.sync_copy(buf_v, out_hbm.at[wid * ROWS_PER_W + c])
            return carry

        lax.fori_loop(0, ROWS_PER_W, batch_row, 0)

    return k(tok2d, table, pos2)


def kernel(tokens, token_table, position_embeddings):
    batch, n_token = tokens.shape
    tok2d = tokens.astype(jnp.int32).reshape(-1, CHUNK)
    pos2 = position_embeddings.reshape(2, CHUNK, N_EMBD)
    out = _emb_call(tok2d, token_table, pos2)
    return out.reshape(batch, n_token, N_EMBD)
